# baseline (speedup 1.0000x reference)
import functools

import jax
import jax.numpy as jnp
from jax import lax
from jax.experimental import pallas as pl
from jax.experimental.pallas import tpu as pltpu
from jax.experimental.pallas import tpu_sc as plsc

VOCAB = 100000
D = 256
DW = D // 2
B, F, L = 1024, 16, 50
NSEG = B * F
ROWS_BLK = 2000
NBLK = 51
VOCAB_PAD = NBLK * ROWS_BLK
NW = 32
SEG_PER_W = NSEG // NW
TOK_PER_W = SEG_PER_W * L
LP = 64
LPAD = 80
NBUF = 4
SEGB = 512


def _ln_table_body(emb_ref, gamma_ref, out_ref):
    i = pl.program_id(0)
    x = emb_ref[...]
    mu = jnp.mean(x, axis=1, keepdims=True)
    var = jnp.mean((x - mu) ** 2, axis=1, keepdims=True)
    y = (x - mu) * lax.rsqrt(var + 1e-5) * gamma_ref[...]
    row = i * ROWS_BLK + lax.broadcasted_iota(jnp.int32, (ROWS_BLK, 1), 0)
    y = jnp.where(row < VOCAB, y, 0.0)
    u = lax.bitcast_convert_type(y, jnp.uint32)
    t = u + jnp.uint32(0x7FFF) + ((u >> 16) & jnp.uint32(1))
    h = t >> 16
    out_ref[...] = h[:, :DW] | (h[:, DW:] << 16)


def _build_table(emb, gamma):
    return pl.pallas_call(
        _ln_table_body,
        grid=(NBLK,),
        in_specs=[
            pl.BlockSpec((ROWS_BLK, D), lambda i: (jnp.minimum(i, NBLK - 2), 0)),
            pl.BlockSpec((1, D), lambda i: (0, 0)),
        ],
        out_specs=pl.BlockSpec((ROWS_BLK, DW), lambda i: (i, 0)),
        out_shape=jax.ShapeDtypeStruct((VOCAB_PAD, DW), jnp.uint32),
    )(emb, gamma)


def _sc_pool(ids_flat, mask_flat, ntab, beta):
    mesh = plsc.VectorSubcoreMesh(core_axis_name="c", subcore_axis_name="s")

    @functools.partial(
        pl.kernel,
        mesh=mesh,
        out_type=jax.ShapeDtypeStruct((NSEG * D,), jnp.float32),
        scratch_types=[
            pltpu.VMEM((TOK_PER_W + 16,), jnp.int32),
            pltpu.VMEM((TOK_PER_W + 16,), jnp.int32),
            pltpu.VMEM((NBUF, LPAD), jnp.int32),
            pltpu.VMEM((NBUF, LP, DW), jnp.uint32),
            pltpu.VMEM((2, D), jnp.float32),
            pltpu.VMEM((D,), jnp.float32),
            pltpu.SemaphoreType.DMA,
            pltpu.SemaphoreType.DMA,
        ],
        compiler_params=pltpu.CompilerParams(needs_layout_passes=False),
    )
    def body(ids_hbm, mask_hbm, ntab_hbm, beta_hbm, out_hbm,
             ids_v, mask_v, cidx_v, gbuf_v, orow_v, beta_v, gsem, osem):
        wid = lax.axis_index("s") * 2 + lax.axis_index("c")
        base = wid * TOK_PER_W
        pltpu.sync_copy(ids_hbm.at[pl.ds(base, TOK_PER_W)],
                        ids_v.at[pl.ds(0, TOK_PER_W)])
        pltpu.sync_copy(mask_hbm.at[pl.ds(base, TOK_PER_W)],
                        mask_v.at[pl.ds(0, TOK_PER_W)])
        pltpu.sync_copy(beta_hbm, beta_v)
        lane = lax.iota(jnp.int32, 16)
        himask = jnp.uint32(0xFFFF0000)

        def fire_gather(s, buf):
            off = s * L
            for j2 in range(LPAD // 16):
                padv = (VOCAB + ((s * LPAD + j2 * 16) & 511)) + lane
                cidx_v[buf, pl.ds(j2 * 16, 16)] = padv
            p = jnp.int32(0)
            for j in range(4):
                o = off + j * 16
                idv = ids_v[pl.ds(o, 16)]
                mv = mask_v[pl.ds(o, 16)]
                valid = mv != 0
                if j == 3:
                    valid = valid & (lane < (L - 48))
                plsc.store_compressed(cidx_v.at[buf, pl.ds(p, 16)], idv,
                                      mask=valid)
                p = p + plsc.all_reduce_population_count(valid)[0]
            nch = (p + 7) >> 3

            def fire(j, carry):
                pltpu.make_async_copy(
                    ntab_hbm.at[cidx_v.at[buf, pl.ds(j * 8, 8)]],
                    gbuf_v.at[buf, pl.ds(j * 8, 8)], gsem).start()
                return carry

            lax.fori_loop(0, nch, fire, 0)
            return nch, p

        pipe0 = tuple(fire_gather(i, i) for i in range(NBUF - 1))

        def seg_body(s, pipe):
            nch_cur, cnt_cur = pipe[0]
            par = lax.rem(s, NBUF)
            st_new = lax.cond(s < SEG_PER_W - (NBUF - 1),
                              lambda: fire_gather(s + (NBUF - 1),
                                                  lax.rem(s + (NBUF - 1),
                                                          NBUF)),
                              lambda: (jnp.int32(0), jnp.int32(0)))

            def chunk_body(j, acc):
                pltpu.make_async_copy(
                    ntab_hbm.at[cidx_v.at[par, pl.ds(0, 8)]],
                    gbuf_v.at[par, pl.ds(0, 8)], gsem).wait()
                new = list(acc)
                for r8 in range(8):
                    r = j * 8 + r8
                    for g in range(8):
                        w = gbuf_v[par, r, pl.ds(g * 16, 16)]
                        new[g] = new[g] + plsc.bitcast(w << 16, jnp.float32)
                        new[8 + g] = new[8 + g] + plsc.bitcast(w & himask,
                                                               jnp.float32)
                return tuple(new)

            zero = jnp.zeros((16,), jnp.float32)
            acc = lax.fori_loop(0, nch_cur, chunk_body, (zero,) * 16)

            opar = lax.rem(s, 2)

            @pl.when(s >= 2)
            def _():
                pltpu.make_async_copy(orow_v.at[opar],
                                      out_hbm.at[pl.ds(0, D)], osem).wait()

            cv = jnp.full((16,), cnt_cur, dtype=jnp.int32)
            cf = cv.astype(jnp.float32)
            has = cv > 0
            inv = jnp.where(has, 1.0 / jnp.maximum(cf, 1.0), 0.0)
            for g in range(8):
                blo = jnp.where(has, beta_v[pl.ds(g * 16, 16)], 0.0)
                bhi = jnp.where(has, beta_v[pl.ds(128 + g * 16, 16)], 0.0)
                orow_v[opar, pl.ds(g * 16, 16)] = acc[g] * inv + blo
                orow_v[opar, pl.ds(128 + g * 16, 16)] = acc[8 + g] * inv + bhi
            gseg = wid * SEG_PER_W + s
            pltpu.make_async_copy(orow_v.at[opar],
                                  out_hbm.at[pl.ds(gseg * D, D)], osem).start()
            return pipe[1:] + (st_new,)

        lax.fori_loop(0, SEG_PER_W, seg_body, pipe0)
        for _ in range(2):
            pltpu.make_async_copy(orow_v.at[0],
                                  out_hbm.at[pl.ds(0, D)], osem).wait()

    return body(ids_flat, mask_flat, ntab, beta)


def kernel(input_ids, attn_mask, emb, gamma, beta):
    ids_flat = input_ids.reshape(-1).astype(jnp.int32)
    mask_flat = attn_mask.reshape(-1).astype(jnp.int32)
    ntab = _build_table(emb, gamma.reshape(1, D))
    pooled = _sc_pool(ids_flat, mask_flat, ntab, beta.reshape(D))
    return pooled.reshape(B, F, D)


# speedup vs baseline: 1.0033x; 1.0033x over previous
import functools

import jax
import jax.numpy as jnp
from jax import lax
from jax.experimental import pallas as pl
from jax.experimental.pallas import tpu as pltpu
from jax.experimental.pallas import tpu_sc as plsc

VOCAB = 100000
D = 256
DW = D // 2
B, F, L = 1024, 16, 50
NSEG = B * F
ROWS_BLK = 2000
NBLK = 51
VOCAB_PAD = NBLK * ROWS_BLK
NW = 32
SEG_PER_W = NSEG // NW
TOK_PER_W = SEG_PER_W * L
LP = 64
LPAD = 80
NBUF = 4
SEGB = 512


def _ln_table_body(emb_ref, gamma_ref, out_ref):
    i = pl.program_id(0)
    x = emb_ref[...]
    mu = jnp.mean(x, axis=1, keepdims=True)
    var = jnp.mean((x - mu) ** 2, axis=1, keepdims=True)
    y = (x - mu) * lax.rsqrt(var + 1e-5) * gamma_ref[...]
    row = i * ROWS_BLK + lax.broadcasted_iota(jnp.int32, (ROWS_BLK, 1), 0)
    y = jnp.where(row < VOCAB, y, 0.0)
    u = lax.bitcast_convert_type(y, jnp.uint32)
    t = u + jnp.uint32(0x7FFF) + ((u >> 16) & jnp.uint32(1))
    h = t >> 16
    out_ref[...] = h[:, :DW] | (h[:, DW:] << 16)


def _build_table(emb, gamma):
    return pl.pallas_call(
        _ln_table_body,
        grid=(NBLK,),
        in_specs=[
            pl.BlockSpec((ROWS_BLK, D), lambda i: (jnp.minimum(i, NBLK - 2), 0)),
            pl.BlockSpec((1, D), lambda i: (0, 0)),
        ],
        out_specs=pl.BlockSpec((ROWS_BLK, DW), lambda i: (i, 0)),
        out_shape=jax.ShapeDtypeStruct((VOCAB_PAD, DW), jnp.uint32),
    )(emb, gamma)


def _sc_pool(ids_flat, mask_flat, ntab, beta):
    mesh = plsc.VectorSubcoreMesh(core_axis_name="c", subcore_axis_name="s")

    @functools.partial(
        pl.kernel,
        mesh=mesh,
        out_type=jax.ShapeDtypeStruct((NSEG * D,), jnp.float32),
        scratch_types=[
            pltpu.VMEM((TOK_PER_W + 16,), jnp.int32),
            pltpu.VMEM((TOK_PER_W + 16,), jnp.int32),
            pltpu.VMEM((NBUF, LPAD), jnp.int32),
            pltpu.VMEM((NBUF, LP, DW), jnp.uint32),
            pltpu.VMEM((2, D), jnp.float32),
            pltpu.VMEM((D,), jnp.float32),
            pltpu.SemaphoreType.DMA,
            pltpu.SemaphoreType.DMA,
        ],
        compiler_params=pltpu.CompilerParams(needs_layout_passes=False),
    )
    def body(ids_hbm, mask_hbm, ntab_hbm, beta_hbm, out_hbm,
             ids_v, mask_v, cidx_v, gbuf_v, orow_v, beta_v, gsem, osem):
        wid = lax.axis_index("s") * 2 + lax.axis_index("c")
        base = wid * TOK_PER_W
        pltpu.sync_copy(ids_hbm.at[pl.ds(base, TOK_PER_W)],
                        ids_v.at[pl.ds(0, TOK_PER_W)])
        pltpu.sync_copy(mask_hbm.at[pl.ds(base, TOK_PER_W)],
                        mask_v.at[pl.ds(0, TOK_PER_W)])
        pltpu.sync_copy(beta_hbm, beta_v)
        lane = lax.iota(jnp.int32, 16)
        himask = jnp.uint32(0xFFFF0000)

        def fire_gather(s, buf):
            off = s * L
            for j2 in range(LPAD // 16):
                padv = (VOCAB + ((s * LPAD + j2 * 16) & 511)) + lane
                cidx_v[buf, pl.ds(j2 * 16, 16)] = padv
            p = jnp.int32(0)
            for j in range(4):
                o = off + j * 16
                idv = ids_v[pl.ds(o, 16)]
                mv = mask_v[pl.ds(o, 16)]
                valid = mv != 0
                if j == 3:
                    valid = valid & (lane < (L - 48))
                plsc.store_compressed(cidx_v.at[buf, pl.ds(p, 16)], idv,
                                      mask=valid)
                p = p + plsc.all_reduce_population_count(valid)[0]
            nch = (p + 7) >> 3

            def fire(j, carry):
                pltpu.make_async_copy(
                    ntab_hbm.at[cidx_v.at[buf, pl.ds(j * 8, 8)]],
                    gbuf_v.at[buf, pl.ds(j * 8, 8)], gsem).start()
                return carry

            lax.fori_loop(0, nch, fire, 0)
            return nch, p

        pipe0 = tuple(fire_gather(i, i) for i in range(NBUF - 1))

        def seg_body(s, pipe):
            nch_cur, cnt_cur = pipe[0]
            par = lax.rem(s, NBUF)
            st_new = lax.cond(s < SEG_PER_W - (NBUF - 1),
                              lambda: fire_gather(s + (NBUF - 1),
                                                  lax.rem(s + (NBUF - 1),
                                                          NBUF)),
                              lambda: (jnp.int32(0), jnp.int32(0)))

            def drain(j, carry):
                pltpu.make_async_copy(
                    ntab_hbm.at[cidx_v.at[par, pl.ds(0, 8)]],
                    gbuf_v.at[par, pl.ds(0, 8)], gsem).wait()
                return carry

            lax.fori_loop(0, nch_cur, drain, 0)

            def acc_body(r, acc):
                new = list(acc)
                for g in range(8):
                    w = gbuf_v[par, r, pl.ds(g * 16, 16)]
                    new[g] = acc[g] + plsc.bitcast(w << 16, jnp.float32)
                    new[8 + g] = acc[8 + g] + plsc.bitcast(w & himask,
                                                           jnp.float32)
                return tuple(new)

            zero = jnp.zeros((16,), jnp.float32)
            acc = lax.fori_loop(0, nch_cur * 8, acc_body, (zero,) * 16)

            opar = lax.rem(s, 2)

            @pl.when(s >= 2)
            def _():
                pltpu.make_async_copy(orow_v.at[opar],
                                      out_hbm.at[pl.ds(0, D)], osem).wait()

            cv = jnp.full((16,), cnt_cur, dtype=jnp.int32)
            cf = cv.astype(jnp.float32)
            has = cv > 0
            inv = jnp.where(has, 1.0 / jnp.maximum(cf, 1.0), 0.0)
            for g in range(8):
                blo = jnp.where(has, beta_v[pl.ds(g * 16, 16)], 0.0)
                bhi = jnp.where(has, beta_v[pl.ds(128 + g * 16, 16)], 0.0)
                orow_v[opar, pl.ds(g * 16, 16)] = acc[g] * inv + blo
                orow_v[opar, pl.ds(128 + g * 16, 16)] = acc[8 + g] * inv + bhi
            gseg = wid * SEG_PER_W + s
            pltpu.make_async_copy(orow_v.at[opar],
                                  out_hbm.at[pl.ds(gseg * D, D)], osem).start()
            return pipe[1:] + (st_new,)

        lax.fori_loop(0, SEG_PER_W, seg_body, pipe0)
        for _ in range(2):
            pltpu.make_async_copy(orow_v.at[0],
                                  out_hbm.at[pl.ds(0, D)], osem).wait()

    return body(ids_flat, mask_flat, ntab, beta)


def kernel(input_ids, attn_mask, emb, gamma, beta):
    ids_flat = input_ids.reshape(-1).astype(jnp.int32)
    mask_flat = attn_mask.reshape(-1).astype(jnp.int32)
    ntab = _build_table(emb, gamma.reshape(1, D))
    pooled = _sc_pool(ids_flat, mask_flat, ntab, beta.reshape(D))
    return pooled.reshape(B, F, D)


# speedup vs baseline: 1.0035x; 1.0002x over previous
import functools

import jax
import jax.numpy as jnp
from jax import lax
from jax.experimental import pallas as pl
from jax.experimental.pallas import tpu as pltpu
from jax.experimental.pallas import tpu_sc as plsc

VOCAB = 100000
D = 256
DW = D // 2
B, F, L = 1024, 16, 50
NSEG = B * F
ROWS_BLK = 2000
NBLK = 51
VOCAB_PAD = NBLK * ROWS_BLK
NW = 32
SEG_PER_W = NSEG // NW
TOK_PER_W = SEG_PER_W * L
LP = 64
LPAD = 80
NBUF = 4
SEGB = 512


def _ln_table_body(emb_ref, gamma_ref, out_ref):
    i = pl.program_id(0)
    x = emb_ref[...]
    mu = jnp.mean(x, axis=1, keepdims=True)
    var = jnp.mean((x - mu) ** 2, axis=1, keepdims=True)
    y = (x - mu) * lax.rsqrt(var + 1e-5) * gamma_ref[...]
    row = i * ROWS_BLK + lax.broadcasted_iota(jnp.int32, (ROWS_BLK, 1), 0)
    y = jnp.where(row < VOCAB, y, 0.0)
    u = lax.bitcast_convert_type(y, jnp.uint32)
    t = u + jnp.uint32(0x7FFF) + ((u >> 16) & jnp.uint32(1))
    h = t >> 16
    out_ref[...] = h[:, :DW] | (h[:, DW:] << 16)


def _build_table(emb, gamma):
    return pl.pallas_call(
        _ln_table_body,
        grid=(NBLK,),
        in_specs=[
            pl.BlockSpec((ROWS_BLK, D), lambda i: (jnp.minimum(i, NBLK - 2), 0)),
            pl.BlockSpec((1, D), lambda i: (0, 0)),
        ],
        out_specs=pl.BlockSpec((ROWS_BLK, DW), lambda i: (i, 0)),
        out_shape=jax.ShapeDtypeStruct((VOCAB_PAD, DW), jnp.uint32),
    )(emb, gamma)


def _sc_pool(ids_flat, mask_flat, ntab, beta):
    mesh = plsc.VectorSubcoreMesh(core_axis_name="c", subcore_axis_name="s")

    @functools.partial(
        pl.kernel,
        mesh=mesh,
        out_type=jax.ShapeDtypeStruct((NSEG * D,), jnp.float32),
        scratch_types=[
            pltpu.VMEM((TOK_PER_W + 16,), jnp.int32),
            pltpu.VMEM((TOK_PER_W + 16,), jnp.int32),
            pltpu.VMEM((NBUF, LPAD), jnp.int32),
            pltpu.VMEM((NBUF, LP, DW), jnp.uint32),
            pltpu.VMEM((2, D), jnp.float32),
            pltpu.VMEM((D,), jnp.float32),
            pltpu.SemaphoreType.DMA,
            pltpu.SemaphoreType.DMA,
        ],
        compiler_params=pltpu.CompilerParams(needs_layout_passes=False,
                                             disable_bounds_checks=True),
    )
    def body(ids_hbm, mask_hbm, ntab_hbm, beta_hbm, out_hbm,
             ids_v, mask_v, cidx_v, gbuf_v, orow_v, beta_v, gsem, osem):
        wid = lax.axis_index("s") * 2 + lax.axis_index("c")
        base = wid * TOK_PER_W
        pltpu.sync_copy(ids_hbm.at[pl.ds(base, TOK_PER_W)],
                        ids_v.at[pl.ds(0, TOK_PER_W)])
        pltpu.sync_copy(mask_hbm.at[pl.ds(base, TOK_PER_W)],
                        mask_v.at[pl.ds(0, TOK_PER_W)])
        pltpu.sync_copy(beta_hbm, beta_v)
        lane = lax.iota(jnp.int32, 16)
        himask = jnp.uint32(0xFFFF0000)

        def fire_gather(s, buf):
            off = s * L
            for j2 in range(LPAD // 16):
                padv = (VOCAB + ((s * LPAD + j2 * 16) & 511)) + lane
                cidx_v[buf, pl.ds(j2 * 16, 16)] = padv
            p = jnp.int32(0)
            for j in range(4):
                o = off + j * 16
                idv = ids_v[pl.ds(o, 16)]
                mv = mask_v[pl.ds(o, 16)]
                valid = mv != 0
                if j == 3:
                    valid = valid & (lane < (L - 48))
                plsc.store_compressed(cidx_v.at[buf, pl.ds(p, 16)], idv,
                                      mask=valid)
                p = p + plsc.all_reduce_population_count(valid)[0]
            nch = (p + 7) >> 3

            def fire(j, carry):
                pltpu.make_async_copy(
                    ntab_hbm.at[cidx_v.at[buf, pl.ds(j * 8, 8)]],
                    gbuf_v.at[buf, pl.ds(j * 8, 8)], gsem).start()
                return carry

            lax.fori_loop(0, nch, fire, 0)
            return nch, p

        pipe0 = tuple(fire_gather(i, i) for i in range(NBUF - 1))

        def seg_body(s, pipe):
            nch_cur, cnt_cur = pipe[0]
            par = lax.rem(s, NBUF)
            st_new = lax.cond(s < SEG_PER_W - (NBUF - 1),
                              lambda: fire_gather(s + (NBUF - 1),
                                                  lax.rem(s + (NBUF - 1),
                                                          NBUF)),
                              lambda: (jnp.int32(0), jnp.int32(0)))

            def drain(j, carry):
                pltpu.make_async_copy(
                    ntab_hbm.at[cidx_v.at[par, pl.ds(0, 8)]],
                    gbuf_v.at[par, pl.ds(0, 8)], gsem).wait()
                return carry

            lax.fori_loop(0, nch_cur, drain, 0)

            def acc_body(r, acc):
                new = list(acc)
                for g in range(8):
                    w = gbuf_v[par, r, pl.ds(g * 16, 16)]
                    new[g] = acc[g] + plsc.bitcast(w << 16, jnp.float32)
                    new[8 + g] = acc[8 + g] + plsc.bitcast(w & himask,
                                                           jnp.float32)
                return tuple(new)

            zero = jnp.zeros((16,), jnp.float32)
            acc = lax.fori_loop(0, nch_cur * 8, acc_body, (zero,) * 16)

            opar = lax.rem(s, 2)

            @pl.when(s >= 2)
            def _():
                pltpu.make_async_copy(orow_v.at[opar],
                                      out_hbm.at[pl.ds(0, D)], osem).wait()

            cv = jnp.full((16,), cnt_cur, dtype=jnp.int32)
            cf = cv.astype(jnp.float32)
            has = cv > 0
            inv = jnp.where(has, 1.0 / jnp.maximum(cf, 1.0), 0.0)
            for g in range(8):
                blo = jnp.where(has, beta_v[pl.ds(g * 16, 16)], 0.0)
                bhi = jnp.where(has, beta_v[pl.ds(128 + g * 16, 16)], 0.0)
                orow_v[opar, pl.ds(g * 16, 16)] = acc[g] * inv + blo
                orow_v[opar, pl.ds(128 + g * 16, 16)] = acc[8 + g] * inv + bhi
            gseg = wid * SEG_PER_W + s
            pltpu.make_async_copy(orow_v.at[opar],
                                  out_hbm.at[pl.ds(gseg * D, D)], osem).start()
            return pipe[1:] + (st_new,)

        lax.fori_loop(0, SEG_PER_W, seg_body, pipe0)
        for _ in range(2):
            pltpu.make_async_copy(orow_v.at[0],
                                  out_hbm.at[pl.ds(0, D)], osem).wait()

    return body(ids_flat, mask_flat, ntab, beta)


def kernel(input_ids, attn_mask, emb, gamma, beta):
    ids_flat = input_ids.reshape(-1).astype(jnp.int32)
    mask_flat = attn_mask.reshape(-1).astype(jnp.int32)
    ntab = _build_table(emb, gamma.reshape(1, D))
    pooled = _sc_pool(ids_flat, mask_flat, ntab, beta.reshape(D))
    return pooled.reshape(B, F, D)


# speedup vs baseline: 1.0076x; 1.0041x over previous
import functools

import jax
import jax.numpy as jnp
from jax import lax
from jax.experimental import pallas as pl
from jax.experimental.pallas import tpu as pltpu
from jax.experimental.pallas import tpu_sc as plsc

VOCAB = 100000
D = 256
DW = D // 2
B, F, L = 1024, 16, 50
NSEG = B * F
ROWS_BLK = 2000
NBLK = 51
VOCAB_PAD = NBLK * ROWS_BLK
NW = 32
SEG_PER_W = NSEG // NW
TOK_PER_W = SEG_PER_W * L
LP = 64
LPAD = 80
NBUF = 4
SEGB = 512


def _ln_table_body(emb_ref, gamma_ref, out_ref):
    i = pl.program_id(0)
    x = emb_ref[...]
    mu = jnp.mean(x, axis=1, keepdims=True)
    var = jnp.mean((x - mu) ** 2, axis=1, keepdims=True)
    y = (x - mu) * lax.rsqrt(var + 1e-5) * gamma_ref[...]
    row = i * ROWS_BLK + lax.broadcasted_iota(jnp.int32, (ROWS_BLK, 1), 0)
    y = jnp.where(row < VOCAB, y, 0.0)
    u = lax.bitcast_convert_type(y, jnp.uint32)
    t = u + jnp.uint32(0x7FFF) + ((u >> 16) & jnp.uint32(1))
    h = t >> 16
    out_ref[...] = h[:, :DW] | (h[:, DW:] << 16)


def _build_table(emb, gamma):
    return pl.pallas_call(
        _ln_table_body,
        grid=(NBLK,),
        in_specs=[
            pl.BlockSpec((ROWS_BLK, D), lambda i: (jnp.minimum(i, NBLK - 2), 0)),
            pl.BlockSpec((1, D), lambda i: (0, 0)),
        ],
        out_specs=pl.BlockSpec((ROWS_BLK, DW), lambda i: (i, 0)),
        out_shape=jax.ShapeDtypeStruct((VOCAB_PAD, DW), jnp.uint32),
    )(emb, gamma)


def _sc_pool(ids_flat, mask_flat, ntab, beta):
    mesh = plsc.VectorSubcoreMesh(core_axis_name="c", subcore_axis_name="s")

    @functools.partial(
        pl.kernel,
        mesh=mesh,
        out_type=jax.ShapeDtypeStruct((NSEG * D,), jnp.float32),
        scratch_types=[
            pltpu.VMEM((TOK_PER_W + 16,), jnp.int32),
            pltpu.VMEM((TOK_PER_W + 16,), jnp.int32),
            pltpu.VMEM((NBUF, LPAD), jnp.int32),
            pltpu.VMEM((NBUF, LP, DW), jnp.uint32),
            pltpu.VMEM((2, D), jnp.float32),
            pltpu.VMEM((D,), jnp.float32),
            pltpu.SemaphoreType.DMA,
            pltpu.SemaphoreType.DMA,
        ],
        compiler_params=pltpu.CompilerParams(needs_layout_passes=False),
    )
    def body(ids_hbm, mask_hbm, ntab_hbm, beta_hbm, out_hbm,
             ids_v, mask_v, cidx_v, gbuf_v, orow_v, beta_v, gsem, osem):
        wid = lax.axis_index("s") * 2 + lax.axis_index("c")
        base = wid * TOK_PER_W
        pltpu.sync_copy(ids_hbm.at[pl.ds(base, TOK_PER_W)],
                        ids_v.at[pl.ds(0, TOK_PER_W)])
        pltpu.sync_copy(mask_hbm.at[pl.ds(base, TOK_PER_W)],
                        mask_v.at[pl.ds(0, TOK_PER_W)])
        pltpu.sync_copy(beta_hbm, beta_v)
        lane = lax.iota(jnp.int32, 16)
        himask = jnp.uint32(0xFFFF0000)

        def fire_gather(s, buf):
            off = s * L
            for j2 in range(LPAD // 16):
                padv = (VOCAB + ((s * LPAD + j2 * 16) & 511)) + lane
                cidx_v[buf, pl.ds(j2 * 16, 16)] = padv
            p = jnp.int32(0)
            for j in range(4):
                o = off + j * 16
                idv = ids_v[pl.ds(o, 16)]
                mv = mask_v[pl.ds(o, 16)]
                valid = mv != 0
                if j == 3:
                    valid = valid & (lane < (L - 48))
                plsc.store_compressed(cidx_v.at[buf, pl.ds(p, 16)], idv,
                                      mask=valid)
                p = p + plsc.all_reduce_population_count(valid)[0]
            nch = (p + 7) >> 3

            def fire(j, carry):
                pltpu.make_async_copy(
                    ntab_hbm.at[cidx_v.at[buf, pl.ds(j * 8, 8)]],
                    gbuf_v.at[buf, pl.ds(j * 8, 8)], gsem).start()
                return carry

            lax.fori_loop(0, nch, fire, 0)
            return nch, p

        pipe0 = tuple(fire_gather(i, i) for i in range(NBUF - 1))

        def seg_body(s, pipe):
            nch_cur, cnt_cur = pipe[0]
            par = lax.rem(s, NBUF)
            st_new = lax.cond(s < SEG_PER_W - (NBUF - 1),
                              lambda: fire_gather(s + (NBUF - 1),
                                                  lax.rem(s + (NBUF - 1),
                                                          NBUF)),
                              lambda: (jnp.int32(0), jnp.int32(0)))

            def drain(j, carry):
                pltpu.make_async_copy(
                    ntab_hbm.at[cidx_v.at[par, pl.ds(0, 8)]],
                    gbuf_v.at[par, pl.ds(0, 8)], gsem).wait()
                return carry

            lax.fori_loop(0, nch_cur, drain, 0)

            def acc_body(r, acc):
                new = list(acc)
                for g in range(8):
                    w = gbuf_v[par, r, pl.ds(g * 16, 16)]
                    new[g] = acc[g] + plsc.bitcast(w << 16, jnp.float32)
                    new[8 + g] = acc[8 + g] + plsc.bitcast(w & himask,
                                                           jnp.float32)
                return tuple(new)

            zero = jnp.zeros((16,), jnp.float32)
            acc = lax.fori_loop(0, nch_cur * 8, acc_body, (zero,) * 16)

            opar = lax.rem(s, 2)

            @pl.when(s >= 2)
            def _():
                pltpu.make_async_copy(orow_v.at[opar],
                                      out_hbm.at[pl.ds(0, D)], osem).wait()

            cv = jnp.full((16,), cnt_cur, dtype=jnp.int32)
            cf = cv.astype(jnp.float32)
            has = cv > 0
            inv = jnp.where(has, 1.0 / jnp.maximum(cf, 1.0), 0.0)
            for g in range(8):
                blo = jnp.where(has, beta_v[pl.ds(g * 16, 16)], 0.0)
                bhi = jnp.where(has, beta_v[pl.ds(128 + g * 16, 16)], 0.0)
                orow_v[opar, pl.ds(g * 16, 16)] = acc[g] * inv + blo
                orow_v[opar, pl.ds(128 + g * 16, 16)] = acc[8 + g] * inv + bhi
            gseg = wid * SEG_PER_W + s
            pltpu.make_async_copy(orow_v.at[opar],
                                  out_hbm.at[pl.ds(gseg * D, D)], osem).start()
            return pipe[1:] + (st_new,)

        lax.fori_loop(0, SEG_PER_W, seg_body, pipe0)
        for _ in range(2):
            pltpu.make_async_copy(orow_v.at[0],
                                  out_hbm.at[pl.ds(0, D)], osem).wait()

    return body(ids_flat, mask_flat, ntab, beta)


def kernel(input_ids, attn_mask, emb, gamma, beta):
    ids_flat = input_ids.reshape(-1).astype(jnp.int32)
    mask_flat = attn_mask.reshape(-1).astype(jnp.int32)
    ntab = _build_table(emb, gamma.reshape(1, D))
    pooled = _sc_pool(ids_flat, mask_flat, ntab, beta.reshape(D))
    return pooled.reshape(B, F, D)
